# baseline (speedup 1.0000x reference)
import functools

import jax
import jax.numpy as jnp
from jax import lax
from jax.experimental import pallas as pl
from jax.experimental.pallas import tpu as pltpu
from jax.experimental.pallas import tpu_sc as plsc

D = 128
NC = 2
NS = 16
NW = NC * NS
CHUNK = 256
NBUF = 2
LANES = 16


def _sc_body(pe_t_h, pe_h_h, pe_w_h, t_h, h_h, w_h, out_h,
             pt_v, ph_v, pw_v, ti_v, hi_v, wi_v, ob_v,
             sit, sih, siw, so):
    wid = lax.axis_index("s") * NC + lax.axis_index("c")
    n = t_h.shape[0]
    per_w = n // NW
    base = wid * per_w
    nch = per_w // CHUNK

    pltpu.sync_copy(pe_t_h, pt_v)
    pltpu.sync_copy(pe_h_h, ph_v)
    pltpu.sync_copy(pe_w_h, pw_v)

    def idx_copies(k, slot):
        off = base + k * CHUNK
        return (
            pltpu.make_async_copy(t_h.at[pl.ds(off, CHUNK)],
                                  ti_v.at[pl.ds(slot * CHUNK, CHUNK)],
                                  sit.at[slot]),
            pltpu.make_async_copy(h_h.at[pl.ds(off, CHUNK)],
                                  hi_v.at[pl.ds(slot * CHUNK, CHUNK)],
                                  sih.at[slot]),
            pltpu.make_async_copy(w_h.at[pl.ds(off, CHUNK)],
                                  wi_v.at[pl.ds(slot * CHUNK, CHUNK)],
                                  siw.at[slot]),
        )

    def out_copy(k, slot):
        off = base + k * CHUNK
        return pltpu.make_async_copy(
            ob_v.at[pl.ds(slot * CHUNK, CHUNK)],
            out_h.at[pl.ds(off, CHUNK)],
            so.at[slot])

    for k0 in range(NBUF):
        for cp in idx_copies(k0, k0):
            cp.start()

    def chunk_body(k, carry):
        slot = k % NBUF
        for cp in idx_copies(k, slot):
            cp.wait()

        @pl.when(k >= NBUF)
        def _():
            out_copy(k - NBUF, slot).wait()

        rbase = slot * CHUNK
        ibase = slot * CHUNK

        @plsc.parallel_loop(0, CHUNK // LANES, 1, unroll=2)
        def group_body(g):
            tids = ti_v[pl.ds(ibase + g * LANES, LANES)]
            hids = hi_v[pl.ds(ibase + g * LANES, LANES)]
            wids = wi_v[pl.ds(ibase + g * LANES, LANES)]
            for j in range(LANES):
                tr = tids[j] * 2
                hr = hids[j] * 2
                wr = wids[j] * 2
                r = rbase + g * LANES + j
                for c in range(D // (2 * LANES)):
                    vb = (pt_v[tr, pl.ds(c * LANES, 2 * LANES)]
                          + ph_v[hr, pl.ds(c * LANES, 2 * LANES)]
                          + pw_v[wr, pl.ds(c * LANES, 2 * LANES)])
                    v0, v1 = plsc.unpack(
                        vb, format=plsc.PackFormat.INTERLEAVED,
                        preferred_element_type=jnp.float32)
                    ob_v[r, pl.ds(c * 2 * LANES, LANES)] = v0
                    ob_v[r, pl.ds(c * 2 * LANES + LANES, LANES)] = v1

        out_copy(k, slot).start()

        @pl.when(k + NBUF < nch)
        def _():
            for cp in idx_copies(k + NBUF, slot):
                cp.start()

        return carry

    lax.fori_loop(0, nch, chunk_body, 0)

    for k0 in range(nch - NBUF, nch):
        out_copy(k0, k0 % NBUF).wait()


def _shuffle_bf16(pe):
    r = pe.shape[0]
    x = pe.reshape(r, D // 32, 2, LANES).transpose(0, 2, 1, 3)
    x = x.reshape(r, 2, D // 2)
    x = jnp.pad(x, ((0, 0), (0, 0), (0, D // 2)))
    return x.reshape(2 * r, D).astype(jnp.bfloat16)


def kernel(pe_t, pe_h, pe_w, t, h, w):
    b, l = t.shape
    n = b * l
    tf = t.reshape(n)
    hf = h.reshape(n)
    wf = w.reshape(n)
    mesh = plsc.VectorSubcoreMesh(core_axis_name="c", subcore_axis_name="s")
    run = pl.kernel(
        _sc_body,
        mesh=mesh,
        compiler_params=pltpu.CompilerParams(needs_layout_passes=False),
        out_type=jax.ShapeDtypeStruct((n, D), jnp.float32),
        scratch_types=[
            pltpu.VMEM((2 * pe_t.shape[0], D), jnp.bfloat16),
            pltpu.VMEM((2 * pe_h.shape[0], D), jnp.bfloat16),
            pltpu.VMEM((2 * pe_w.shape[0], D), jnp.bfloat16),
            pltpu.VMEM((NBUF * CHUNK,), jnp.int32),
            pltpu.VMEM((NBUF * CHUNK,), jnp.int32),
            pltpu.VMEM((NBUF * CHUNK,), jnp.int32),
            pltpu.VMEM((NBUF * CHUNK, D), jnp.float32),
            pltpu.SemaphoreType.DMA((NBUF,)),
            pltpu.SemaphoreType.DMA((NBUF,)),
            pltpu.SemaphoreType.DMA((NBUF,)),
            pltpu.SemaphoreType.DMA((NBUF,)),
        ],
    )
    out = run(_shuffle_bf16(pe_t), _shuffle_bf16(pe_h), _shuffle_bf16(pe_w),
              tf, hf, wf)
    return out.reshape(b, l, D)


# speedup vs baseline: 1.9810x; 1.9810x over previous
import functools

import jax
import jax.numpy as jnp
from jax import lax
from jax.experimental import pallas as pl
from jax.experimental.pallas import tpu as pltpu
from jax.experimental.pallas import tpu_sc as plsc

D = 128
NC = 2
NS = 16
NW = NC * NS
CHUNK = 256
NBUF = 2
LANES = 16


def _sc_body(pe_t_h, pe_h_h, pe_w_h, t_h, h_h, w_h, out_h,
             pt_v, ph_v, pw_v, ti_v, hi_v, wi_v, ob_v,
             sit, sih, siw, so):
    wid = lax.axis_index("s") * NC + lax.axis_index("c")
    n = t_h.shape[0]
    per_w = n // NW
    base = wid * per_w
    nch = per_w // CHUNK

    pltpu.sync_copy(pe_t_h, pt_v)
    pltpu.sync_copy(pe_h_h, ph_v)
    pltpu.sync_copy(pe_w_h, pw_v)

    def idx_copies(k, slot):
        off = base + k * CHUNK
        return (
            pltpu.make_async_copy(t_h.at[pl.ds(off, CHUNK)],
                                  ti_v.at[pl.ds(slot * CHUNK, CHUNK)],
                                  sit.at[slot]),
            pltpu.make_async_copy(h_h.at[pl.ds(off, CHUNK)],
                                  hi_v.at[pl.ds(slot * CHUNK, CHUNK)],
                                  sih.at[slot]),
            pltpu.make_async_copy(w_h.at[pl.ds(off, CHUNK)],
                                  wi_v.at[pl.ds(slot * CHUNK, CHUNK)],
                                  siw.at[slot]),
        )

    def out_copy(k, slot):
        off = base + k * CHUNK
        return pltpu.make_async_copy(
            ob_v.at[pl.ds(slot * CHUNK, CHUNK)],
            out_h.at[pl.ds(off, CHUNK)],
            so.at[slot])

    for k0 in range(NBUF):
        for cp in idx_copies(k0, k0):
            cp.start()

    def chunk_body(k, carry):
        slot = k % NBUF
        for cp in idx_copies(k, slot):
            cp.wait()

        @pl.when(k >= NBUF)
        def _():
            out_copy(k - NBUF, slot).wait()

        rbase = slot * CHUNK
        ibase = slot * CHUNK

        @plsc.parallel_loop(0, CHUNK // LANES, 1)
        def group_body(g):
            tids = ti_v[pl.ds(ibase + g * LANES, LANES)]
            hids = hi_v[pl.ds(ibase + g * LANES, LANES)]
            wids = wi_v[pl.ds(ibase + g * LANES, LANES)]
            for j in range(LANES):
                tr = tids[j] * 2
                hr = hids[j] * 2
                wr = wids[j] * 2
                r = rbase + g * LANES + j
                for c in range(D // (2 * LANES)):
                    vb = (pt_v[tr, pl.ds(c * LANES, 2 * LANES)]
                          + ph_v[hr, pl.ds(c * LANES, 2 * LANES)]
                          + pw_v[wr, pl.ds(c * LANES, 2 * LANES)])
                    v0, v1 = plsc.unpack(
                        vb, format=plsc.PackFormat.INTERLEAVED,
                        preferred_element_type=jnp.float32)
                    ob_v[r, pl.ds(c * 2 * LANES, LANES)] = v0
                    ob_v[r, pl.ds(c * 2 * LANES + LANES, LANES)] = v1

        out_copy(k, slot).start()

        @pl.when(k + NBUF < nch)
        def _():
            for cp in idx_copies(k + NBUF, slot):
                cp.start()

        return carry

    lax.fori_loop(0, nch, chunk_body, 0)

    for k0 in range(nch - NBUF, nch):
        out_copy(k0, k0 % NBUF).wait()


def _shuffle_bf16(pe):
    r = pe.shape[0]
    x = pe.reshape(r, D // 32, 2, LANES).transpose(0, 2, 1, 3)
    x = x.reshape(r, 2, D // 2)
    x = jnp.pad(x, ((0, 0), (0, 0), (0, D // 2)))
    return x.reshape(2 * r, D).astype(jnp.bfloat16)


def kernel(pe_t, pe_h, pe_w, t, h, w):
    b, l = t.shape
    n = b * l
    tf = t.reshape(n)
    hf = h.reshape(n)
    wf = w.reshape(n)
    mesh = plsc.VectorSubcoreMesh(core_axis_name="c", subcore_axis_name="s")
    run = pl.kernel(
        _sc_body,
        mesh=mesh,
        compiler_params=pltpu.CompilerParams(needs_layout_passes=False),
        out_type=jax.ShapeDtypeStruct((n, D), jnp.float32),
        scratch_types=[
            pltpu.VMEM((2 * pe_t.shape[0], D), jnp.bfloat16),
            pltpu.VMEM((2 * pe_h.shape[0], D), jnp.bfloat16),
            pltpu.VMEM((2 * pe_w.shape[0], D), jnp.bfloat16),
            pltpu.VMEM((NBUF * CHUNK,), jnp.int32),
            pltpu.VMEM((NBUF * CHUNK,), jnp.int32),
            pltpu.VMEM((NBUF * CHUNK,), jnp.int32),
            pltpu.VMEM((NBUF * CHUNK, D), jnp.float32),
            pltpu.SemaphoreType.DMA((NBUF,)),
            pltpu.SemaphoreType.DMA((NBUF,)),
            pltpu.SemaphoreType.DMA((NBUF,)),
            pltpu.SemaphoreType.DMA((NBUF,)),
        ],
    )
    out = run(_shuffle_bf16(pe_t), _shuffle_bf16(pe_h), _shuffle_bf16(pe_w),
              tf, hf, wf)
    return out.reshape(b, l, D)


# speedup vs baseline: 2.0158x; 1.0175x over previous
import functools

import jax
import jax.numpy as jnp
from jax import lax
from jax.experimental import pallas as pl
from jax.experimental.pallas import tpu as pltpu
from jax.experimental.pallas import tpu_sc as plsc

D = 128
NC = 2
NS = 16
NW = NC * NS
CHUNK = 256
NBUF = 2
LANES = 16


def _sc_body(pe_t_h, pe_h_h, pe_w_h, p_h, out_h,
             pt_v, ph_v, pw_v, pi_v, ob_v, si, so):
    wid = lax.axis_index("s") * NC + lax.axis_index("c")
    n = p_h.shape[0]
    per_w = n // NW
    base = wid * per_w
    nch = per_w // CHUNK

    pltpu.sync_copy(pe_t_h, pt_v)
    pltpu.sync_copy(pe_h_h, ph_v)
    pltpu.sync_copy(pe_w_h, pw_v)

    def idx_copy(k, slot):
        off = base + k * CHUNK
        return pltpu.make_async_copy(p_h.at[pl.ds(off, CHUNK)],
                                     pi_v.at[pl.ds(slot * CHUNK, CHUNK)],
                                     si.at[slot])

    def out_copy(k, slot):
        off = base + k * CHUNK
        return pltpu.make_async_copy(
            ob_v.at[pl.ds(slot * CHUNK, CHUNK)],
            out_h.at[pl.ds(off, CHUNK)],
            so.at[slot])

    for k0 in range(NBUF):
        idx_copy(k0, k0).start()

    def chunk_body(k, carry):
        slot = k % NBUF
        idx_copy(k, slot).wait()

        @pl.when(k >= NBUF)
        def _():
            out_copy(k - NBUF, slot).wait()

        rbase = slot * CHUNK

        @plsc.parallel_loop(0, CHUNK // LANES, 1)
        def group_body(g):
            pids = pi_v[pl.ds(rbase + g * LANES, LANES)]
            for j in range(LANES):
                p = pids[j]
                tr = (p >> 11) & 0x1FE
                hr = (p >> 5) & 0x7E
                wr = (p << 1) & 0x7E
                r = rbase + g * LANES + j
                for c in range(D // (2 * LANES)):
                    vb = (pt_v[tr, pl.ds(c * LANES, 2 * LANES)]
                          + ph_v[hr, pl.ds(c * LANES, 2 * LANES)]
                          + pw_v[wr, pl.ds(c * LANES, 2 * LANES)])
                    v0, v1 = plsc.unpack(
                        vb, format=plsc.PackFormat.INTERLEAVED,
                        preferred_element_type=jnp.float32)
                    ob_v[r, pl.ds(c * 2 * LANES, LANES)] = v0
                    ob_v[r, pl.ds(c * 2 * LANES + LANES, LANES)] = v1

        out_copy(k, slot).start()

        @pl.when(k + NBUF < nch)
        def _():
            idx_copy(k + NBUF, slot).start()

        return carry

    lax.fori_loop(0, nch, chunk_body, 0)

    for k0 in range(nch - NBUF, nch):
        out_copy(k0, k0 % NBUF).wait()


def _shuffle_bf16(pe):
    r = pe.shape[0]
    x = pe.reshape(r, D // 32, 2, LANES).transpose(0, 2, 1, 3)
    x = x.reshape(r, 2, D // 2)
    x = jnp.pad(x, ((0, 0), (0, 0), (0, D // 2)))
    return x.reshape(2 * r, D).astype(jnp.bfloat16)


def kernel(pe_t, pe_h, pe_w, t, h, w):
    b, l = t.shape
    n = b * l
    packed = ((t.astype(jnp.int32) << 12) | (h.astype(jnp.int32) << 6)
              | w.astype(jnp.int32)).reshape(n)
    mesh = plsc.VectorSubcoreMesh(core_axis_name="c", subcore_axis_name="s")
    run = pl.kernel(
        _sc_body,
        mesh=mesh,
        compiler_params=pltpu.CompilerParams(needs_layout_passes=False),
        out_type=jax.ShapeDtypeStruct((n, D), jnp.float32),
        scratch_types=[
            pltpu.VMEM((2 * pe_t.shape[0], D), jnp.bfloat16),
            pltpu.VMEM((2 * pe_h.shape[0], D), jnp.bfloat16),
            pltpu.VMEM((2 * pe_w.shape[0], D), jnp.bfloat16),
            pltpu.VMEM((NBUF * CHUNK,), jnp.int32),
            pltpu.VMEM((NBUF * CHUNK, D), jnp.float32),
            pltpu.SemaphoreType.DMA((NBUF,)),
            pltpu.SemaphoreType.DMA((NBUF,)),
        ],
    )
    out = run(_shuffle_bf16(pe_t), _shuffle_bf16(pe_h), _shuffle_bf16(pe_w),
              packed)
    return out.reshape(b, l, D)


# speedup vs baseline: 2.2986x; 1.1403x over previous
import functools

import jax
import jax.numpy as jnp
from jax import lax
from jax.experimental import pallas as pl
from jax.experimental.pallas import tpu as pltpu
from jax.experimental.pallas import tpu_sc as plsc

D = 128
NC = 2
NS = 16
NW = NC * NS
CHUNK = 320
NBUF = 2
LANES = 16


def _sc_body(pe_t_h, pe_h_h, pe_w_h, p_h, out_h,
             pt_v, ph_v, pw_v, pi_v, ob_v, si, so):
    wid = lax.axis_index("s") * NC + lax.axis_index("c")
    n = p_h.shape[0]
    per_w = n // NW
    base = wid * per_w
    nch = per_w // CHUNK

    pltpu.sync_copy(pe_t_h, pt_v)
    pltpu.sync_copy(pe_h_h, ph_v)
    pltpu.sync_copy(pe_w_h, pw_v)

    def idx_copy(k, slot):
        off = base + k * CHUNK
        return pltpu.make_async_copy(p_h.at[pl.ds(off, CHUNK)],
                                     pi_v.at[pl.ds(slot * CHUNK, CHUNK)],
                                     si.at[slot])

    def out_copy(k, slot):
        off = base + k * CHUNK
        return pltpu.make_async_copy(
            ob_v.at[pl.ds(slot * CHUNK, CHUNK)],
            out_h.at[pl.ds(off, CHUNK)],
            so.at[slot])

    for k0 in range(NBUF):
        idx_copy(k0, k0).start()

    def chunk_body(k, carry):
        slot = k % NBUF
        idx_copy(k, slot).wait()

        @pl.when(k >= NBUF)
        def _():
            out_copy(k - NBUF, slot).wait()

        rbase = slot * CHUNK

        @plsc.parallel_loop(0, CHUNK // LANES, 1)
        def group_body(g):
            pids = pi_v[pl.ds(rbase + g * LANES, LANES)]
            for j in range(LANES):
                p = pids[j]
                tr = (p >> 11) & 0x1FE
                hr = (p >> 5) & 0x7E
                wr = (p << 1) & 0x7E
                r = rbase + g * LANES + j
                for c in range(D // (2 * LANES)):
                    vb = (pt_v[tr, pl.ds(c * LANES, 2 * LANES)]
                          + ph_v[hr, pl.ds(c * LANES, 2 * LANES)]
                          + pw_v[wr, pl.ds(c * LANES, 2 * LANES)])
                    v0, v1 = plsc.unpack(
                        vb, format=plsc.PackFormat.INTERLEAVED,
                        preferred_element_type=jnp.float32)
                    ob_v[r, pl.ds(c * 2 * LANES, LANES)] = v0
                    ob_v[r, pl.ds(c * 2 * LANES + LANES, LANES)] = v1

        out_copy(k, slot).start()

        @pl.when(k + NBUF < nch)
        def _():
            idx_copy(k + NBUF, slot).start()

        return carry

    lax.fori_loop(0, nch, chunk_body, 0)

    for k0 in range(nch - NBUF, nch):
        out_copy(k0, k0 % NBUF).wait()


def _shuffle_bf16(pe):
    r = pe.shape[0]
    x = pe.reshape(r, D // 32, 2, LANES).transpose(0, 2, 1, 3)
    x = x.reshape(r, 2, D // 2)
    x = jnp.pad(x, ((0, 0), (0, 0), (0, D // 2)))
    return x.reshape(2 * r, D).astype(jnp.bfloat16)


def kernel(pe_t, pe_h, pe_w, t, h, w):
    b, l = t.shape
    n = b * l
    packed = ((t.astype(jnp.int32) << 12) | (h.astype(jnp.int32) << 6)
              | w.astype(jnp.int32)).reshape(n)
    mesh = plsc.VectorSubcoreMesh(core_axis_name="c", subcore_axis_name="s")
    run = pl.kernel(
        _sc_body,
        mesh=mesh,
        compiler_params=pltpu.CompilerParams(needs_layout_passes=False),
        out_type=jax.ShapeDtypeStruct((n, D), jnp.float32),
        scratch_types=[
            pltpu.VMEM((2 * pe_t.shape[0], D), jnp.bfloat16),
            pltpu.VMEM((2 * pe_h.shape[0], D), jnp.bfloat16),
            pltpu.VMEM((2 * pe_w.shape[0], D), jnp.bfloat16),
            pltpu.VMEM((NBUF * CHUNK,), jnp.int32),
            pltpu.VMEM((NBUF * CHUNK, D), jnp.float32),
            pltpu.SemaphoreType.DMA((NBUF,)),
            pltpu.SemaphoreType.DMA((NBUF,)),
        ],
    )
    out = run(_shuffle_bf16(pe_t), _shuffle_bf16(pe_h), _shuffle_bf16(pe_w),
              packed)
    return out.reshape(b, l, D)


# speedup vs baseline: 2.3043x; 1.0025x over previous
import functools

import jax
import jax.numpy as jnp
from jax import lax
from jax.experimental import pallas as pl
from jax.experimental.pallas import tpu as pltpu
from jax.experimental.pallas import tpu_sc as plsc

D = 128
NC = 2
NS = 16
NW = NC * NS
CHUNK = 320
NBUF = 2
LANES = 16


def _sc_body(pe_t_h, pe_h_h, pe_w_h, p_h, out_h,
             pt_v, ph_v, pw_v, pi_v, ob_v, si, so):
    wid = lax.axis_index("s") * NC + lax.axis_index("c")
    n = p_h.shape[0]
    per_w = n // NW
    base = wid * per_w
    nch = per_w // CHUNK

    pltpu.sync_copy(pe_t_h, pt_v)
    pltpu.sync_copy(pe_h_h, ph_v)
    pltpu.sync_copy(pe_w_h, pw_v)

    def idx_copy(k, slot):
        off = base + k * CHUNK
        return pltpu.make_async_copy(p_h.at[pl.ds(off, CHUNK)],
                                     pi_v.at[pl.ds(slot * CHUNK, CHUNK)],
                                     si.at[slot])

    def out_copy(k, slot):
        off = base + k * CHUNK
        return pltpu.make_async_copy(
            ob_v.at[pl.ds(slot * CHUNK, CHUNK)],
            out_h.at[pl.ds(off, CHUNK)],
            so.at[slot])

    for k0 in range(NBUF):
        idx_copy(k0, k0).start()

    def chunk_body(k, carry):
        slot = k % NBUF
        idx_copy(k, slot).wait()

        @pl.when(k >= NBUF)
        def _():
            out_copy(k - NBUF, slot).wait()

        rbase = slot * CHUNK

        @plsc.parallel_loop(0, CHUNK // LANES, 1)
        def group_body(g):
            pids = pi_v[pl.ds(rbase + g * LANES, LANES)]
            trs, hrs, wrs = [], [], []
            for j in range(LANES):
                p = pids[j]
                trs.append((p >> 11) & 0x1FE)
                hrs.append((p >> 5) & 0x7E)
                wrs.append((p << 1) & 0x7E)
            for j in range(LANES):
                tr = trs[j]
                hr = hrs[j]
                wr = wrs[j]
                r = rbase + g * LANES + j
                for c in range(D // (2 * LANES)):
                    vb = (pt_v[tr, pl.ds(c * LANES, 2 * LANES)]
                          + ph_v[hr, pl.ds(c * LANES, 2 * LANES)]
                          + pw_v[wr, pl.ds(c * LANES, 2 * LANES)])
                    v0, v1 = plsc.unpack(
                        vb, format=plsc.PackFormat.INTERLEAVED,
                        preferred_element_type=jnp.float32)
                    ob_v[r, pl.ds(c * 2 * LANES, LANES)] = v0
                    ob_v[r, pl.ds(c * 2 * LANES + LANES, LANES)] = v1

        out_copy(k, slot).start()

        @pl.when(k + NBUF < nch)
        def _():
            idx_copy(k + NBUF, slot).start()

        return carry

    lax.fori_loop(0, nch, chunk_body, 0)

    for k0 in range(nch - NBUF, nch):
        out_copy(k0, k0 % NBUF).wait()


def _shuffle_bf16(pe):
    r = pe.shape[0]
    x = pe.reshape(r, D // 32, 2, LANES).transpose(0, 2, 1, 3)
    x = x.reshape(r, 2, D // 2)
    x = jnp.pad(x, ((0, 0), (0, 0), (0, D // 2)))
    return x.reshape(2 * r, D).astype(jnp.bfloat16)


def kernel(pe_t, pe_h, pe_w, t, h, w):
    b, l = t.shape
    n = b * l
    packed = ((t.astype(jnp.int32) << 12) | (h.astype(jnp.int32) << 6)
              | w.astype(jnp.int32)).reshape(n)
    mesh = plsc.VectorSubcoreMesh(core_axis_name="c", subcore_axis_name="s")
    run = pl.kernel(
        _sc_body,
        mesh=mesh,
        compiler_params=pltpu.CompilerParams(needs_layout_passes=False),
        out_type=jax.ShapeDtypeStruct((n, D), jnp.float32),
        scratch_types=[
            pltpu.VMEM((2 * pe_t.shape[0], D), jnp.bfloat16),
            pltpu.VMEM((2 * pe_h.shape[0], D), jnp.bfloat16),
            pltpu.VMEM((2 * pe_w.shape[0], D), jnp.bfloat16),
            pltpu.VMEM((NBUF * CHUNK,), jnp.int32),
            pltpu.VMEM((NBUF * CHUNK, D), jnp.float32),
            pltpu.SemaphoreType.DMA((NBUF,)),
            pltpu.SemaphoreType.DMA((NBUF,)),
        ],
    )
    out = run(_shuffle_bf16(pe_t), _shuffle_bf16(pe_h), _shuffle_bf16(pe_w),
              packed)
    return out.reshape(b, l, D)


# speedup vs baseline: 3.2048x; 1.3908x over previous
import functools

import jax
import jax.numpy as jnp
from jax import lax
from jax.experimental import pallas as pl
from jax.experimental.pallas import tpu as pltpu
from jax.experimental.pallas import tpu_sc as plsc

D = 128
NC = 2
NS = 16
NW = NC * NS
CHUNK = 320
NBUF = 2
LANES = 16


def _sc_body(pe_t_h, pe_h_h, pe_w_h, p_h, out_h,
             pt_v, ph_v, pw_v, pi_v, ob_v, si, so):
    wid = lax.axis_index("s") * NC + lax.axis_index("c")
    n = p_h.shape[0]
    per_w = n // NW
    base = wid * per_w
    nch = per_w // CHUNK

    pltpu.sync_copy(pe_t_h, pt_v)
    pltpu.sync_copy(pe_h_h, ph_v)
    pltpu.sync_copy(pe_w_h, pw_v)

    def idx_copy(k, slot):
        off = base + k * CHUNK
        return pltpu.make_async_copy(p_h.at[pl.ds(off, CHUNK)],
                                     pi_v.at[pl.ds(slot * CHUNK, CHUNK)],
                                     si.at[slot])

    def out_copy(k, slot):
        off = base + k * CHUNK
        return pltpu.make_async_copy(
            ob_v.at[pl.ds(slot * CHUNK, CHUNK)],
            out_h.at[pl.ds(off, CHUNK)],
            so.at[slot])

    for k0 in range(NBUF):
        idx_copy(k0, k0).start()

    def chunk_body(k, carry):
        slot = k % NBUF
        idx_copy(k, slot).wait()

        @pl.when(k >= NBUF)
        def _():
            out_copy(k - NBUF, slot).wait()

        rbase = slot * CHUNK

        @plsc.parallel_loop(0, CHUNK, 1)
        def row_body(j):
            pids = pi_v[pl.ds(rbase + j, LANES)]
            p = pids[0]
            tr = (p >> 11) & 0x1FE
            hr = (p >> 5) & 0x7E
            wr = (p << 1) & 0x7E
            r = rbase + j
            for c in range(D // (2 * LANES)):
                vb = (pt_v[tr, pl.ds(c * LANES, 2 * LANES)]
                      + ph_v[hr, pl.ds(c * LANES, 2 * LANES)]
                      + pw_v[wr, pl.ds(c * LANES, 2 * LANES)])
                v0, v1 = plsc.unpack(
                    vb, format=plsc.PackFormat.INTERLEAVED,
                    preferred_element_type=jnp.float32)
                ob_v[r, pl.ds(c * 2 * LANES, LANES)] = v0
                ob_v[r, pl.ds(c * 2 * LANES + LANES, LANES)] = v1

        out_copy(k, slot).start()

        @pl.when(k + NBUF < nch)
        def _():
            idx_copy(k + NBUF, slot).start()

        return carry

    lax.fori_loop(0, nch, chunk_body, 0)

    for k0 in range(nch - NBUF, nch):
        out_copy(k0, k0 % NBUF).wait()


def _shuffle_bf16(pe):
    r = pe.shape[0]
    x = pe.reshape(r, D // 32, 2, LANES).transpose(0, 2, 1, 3)
    x = x.reshape(r, 2, D // 2)
    x = jnp.pad(x, ((0, 0), (0, 0), (0, D // 2)))
    return x.reshape(2 * r, D).astype(jnp.bfloat16)


def kernel(pe_t, pe_h, pe_w, t, h, w):
    b, l = t.shape
    n = b * l
    packed = ((t.astype(jnp.int32) << 12) | (h.astype(jnp.int32) << 6)
              | w.astype(jnp.int32)).reshape(n)
    mesh = plsc.VectorSubcoreMesh(core_axis_name="c", subcore_axis_name="s")
    run = pl.kernel(
        _sc_body,
        mesh=mesh,
        compiler_params=pltpu.CompilerParams(needs_layout_passes=False),
        out_type=jax.ShapeDtypeStruct((n, D), jnp.float32),
        scratch_types=[
            pltpu.VMEM((2 * pe_t.shape[0], D), jnp.bfloat16),
            pltpu.VMEM((2 * pe_h.shape[0], D), jnp.bfloat16),
            pltpu.VMEM((2 * pe_w.shape[0], D), jnp.bfloat16),
            pltpu.VMEM((NBUF * CHUNK + LANES,), jnp.int32),
            pltpu.VMEM((NBUF * CHUNK, D), jnp.float32),
            pltpu.SemaphoreType.DMA((NBUF,)),
            pltpu.SemaphoreType.DMA((NBUF,)),
        ],
    )
    out = run(_shuffle_bf16(pe_t), _shuffle_bf16(pe_h), _shuffle_bf16(pe_w),
              packed)
    return out.reshape(b, l, D)
